# Initial kernel scaffold; baseline (speedup 1.0000x reference)
import jax
import jax.numpy as jnp
from jax.experimental import pallas as pl


def kernel(x, edge_index, W_enc, W_pair, w_ext, beta, W_dec):
    raise NotImplementedError("write your pallas kernel here")


# speedup vs baseline: 7.7340x; 7.7340x over previous
import functools

import jax
import jax.numpy as jnp
from jax import lax
from jax.experimental import pallas as pl
from jax.experimental.pallas import tpu as pltpu
from jax.experimental.pallas import tpu_sc as plsc

_N = 10000
_E = 320000
_D = 128
_H = 128
_C = 40
_L = 4
_STEP = 0.1

_NC = 2
_NS = 16
_NW = _NC * _NS
_EPW = _E // _NW
_CH = 80
_NCHUNK = _EPW // _CH
_RC = 624
_ZC = 208
_TAIL = _N - _NS * _RC

_R = 1000
_NB = _N // _R


def _sc_degree(col):
    mesh = plsc.VectorSubcoreMesh(core_axis_name="c", subcore_axis_name="s")

    @functools.partial(
        pl.kernel,
        out_type=jax.ShapeDtypeStruct((_NC, _N, _H), jnp.float32),
        mesh=mesh,
        scratch_types=[
            pltpu.VMEM((_CH,), jnp.int32),
            pltpu.VMEM((_CH, _H), jnp.float32),
            pltpu.VMEM((_ZC, _H), jnp.float32),
            pltpu.VMEM_SHARED((_N, _H), jnp.float32),
        ],
    )
    def k(col_hbm, out_hbm, cidx, ones_v, zbuf, shared):
        c = lax.axis_index("c")
        s = lax.axis_index("s")
        wid = c * _NS + s

        @pl.loop(0, _CH)
        def _(i):
            @pl.loop(0, _H, step=16)
            def _(j):
                ones_v[i, pl.ds(j, 16)] = jnp.full((16,), 1.0, jnp.float32)

        @pl.loop(0, _ZC)
        def _(i):
            @pl.loop(0, _H, step=16)
            def _(j):
                zbuf[i, pl.ds(j, 16)] = jnp.zeros((16,), jnp.float32)

        @pl.loop(0, _RC // _ZC)
        def _(t):
            pltpu.sync_copy(zbuf, shared.at[pl.ds(s * _RC + t * _ZC, _ZC)])

        @pl.when(s == _NS - 1)
        def _():
            pltpu.sync_copy(zbuf.at[pl.ds(0, _TAIL)],
                            shared.at[pl.ds(_NS * _RC, _TAIL)])

        plsc.subcore_barrier()

        @pl.loop(0, _NCHUNK)
        def _(i):
            pltpu.sync_copy(col_hbm.at[pl.ds(wid * _EPW + i * _CH, _CH)], cidx)
            pltpu.sync_copy(ones_v, shared.at[cidx], add=True)

        plsc.subcore_barrier()

        @pl.loop(0, _RC // _ZC)
        def _(t):
            pltpu.sync_copy(
                shared.at[pl.ds(s * _RC + t * _ZC, _ZC)],
                out_hbm.at[c, pl.ds(s * _RC + t * _ZC, _ZC)],
            )

        @pl.when(s == _NS - 1)
        def _():
            pltpu.sync_copy(shared.at[pl.ds(_NS * _RC, _TAIL)],
                            out_hbm.at[c, pl.ds(_NS * _RC, _TAIL)])

    return k(col)


def _sc_gather_scatter(g, row, col):
    mesh = plsc.VectorSubcoreMesh(core_axis_name="c", subcore_axis_name="s")

    @functools.partial(
        pl.kernel,
        out_type=jax.ShapeDtypeStruct((_NC, _N, _H), jnp.float32),
        mesh=mesh,
        scratch_types=[
            pltpu.VMEM((_CH,), jnp.int32),
            pltpu.VMEM((_CH,), jnp.int32),
            pltpu.VMEM((_CH, _H), jnp.float32),
            pltpu.VMEM((_ZC, _H), jnp.float32),
            pltpu.VMEM_SHARED((_N, _H), jnp.float32),
            pltpu.SemaphoreType.DMA,
        ],
    )
    def k(g_hbm, row_hbm, col_hbm, out_hbm, ridx, cidx, rows_v, zbuf, shared, sem):
        c = lax.axis_index("c")
        s = lax.axis_index("s")
        wid = c * _NS + s

        @pl.loop(0, _ZC)
        def _(i):
            @pl.loop(0, _H, step=16)
            def _(j):
                zbuf[i, pl.ds(j, 16)] = jnp.zeros((16,), jnp.float32)

        @pl.loop(0, _RC // _ZC)
        def _(t):
            pltpu.sync_copy(zbuf, shared.at[pl.ds(s * _RC + t * _ZC, _ZC)])

        @pl.when(s == _NS - 1)
        def _():
            pltpu.sync_copy(zbuf.at[pl.ds(0, _TAIL)],
                            shared.at[pl.ds(_NS * _RC, _TAIL)])

        plsc.subcore_barrier()

        @pl.loop(0, _NCHUNK)
        def _(i):
            base = wid * _EPW + i * _CH
            pltpu.sync_copy(row_hbm.at[pl.ds(base, _CH)], ridx)
            pltpu.sync_copy(col_hbm.at[pl.ds(base, _CH)], cidx)
            pltpu.async_copy(g_hbm.at[ridx], rows_v, sem).wait()
            pltpu.sync_copy(rows_v, shared.at[cidx], add=True)

        plsc.subcore_barrier()

        @pl.loop(0, _RC // _ZC)
        def _(t):
            pltpu.sync_copy(
                shared.at[pl.ds(s * _RC + t * _ZC, _ZC)],
                out_hbm.at[c, pl.ds(s * _RC + t * _ZC, _ZC)],
            )

        @pl.when(s == _NS - 1)
        def _():
            pltpu.sync_copy(shared.at[pl.ds(_NS * _RC, _TAIL)],
                            out_hbm.at[c, pl.ds(_NS * _RC, _TAIL)])

    return k(g, row, col)


def _wp_compute(W_pair, W_pair_T):
    def body(wp_ref, wpt_ref, out_ref):
        w = wp_ref[...]
        a_t = wpt_ref[...]
        a = w[:, :_H]
        q = w[:, _H:_H + 1]
        r = w[:, _H + 1:_H + 2]
        rows = lax.broadcasted_iota(jnp.int32, (_H, _H), 0)
        cols = lax.broadcasted_iota(jnp.int32, (_H, _H), 1)
        upper = jnp.where(cols > rows, a, 0.0)
        lower = jnp.where(rows > cols, a_t, 0.0)
        w0 = upper + lower
        sumabs = jnp.sum(jnp.abs(w0), axis=1, keepdims=True)
        diagv = q * sumabs + r
        out_ref[...] = w0 + jnp.where(rows == cols, diagv, 0.0)

    return pl.pallas_call(
        body,
        out_shape=jax.ShapeDtypeStruct((_H, _H), jnp.float32),
    )(W_pair, W_pair_T)


def _encode(x, W_enc, Wp, deg2):
    def body(x_ref, we_ref, wp_ref, deg_ref, h_ref, g_ref, dinv_ref):
        db = deg_ref[...]
        d = db[0, :, 0:1] + db[1, :, 0:1]
        dinv = jnp.where(d > 0, lax.rsqrt(d), 0.0)
        h = jnp.dot(x_ref[...], we_ref[...], preferred_element_type=jnp.float32)
        g = dinv * jnp.dot(h, wp_ref[...], preferred_element_type=jnp.float32)
        h_ref[...] = h
        g_ref[...] = g
        dinv_ref[...] = dinv

    return pl.pallas_call(
        body,
        grid=(_NB,),
        in_specs=[
            pl.BlockSpec((_R, _D), lambda i: (i, 0)),
            pl.BlockSpec((_D, _H), lambda i: (0, 0)),
            pl.BlockSpec((_H, _H), lambda i: (0, 0)),
            pl.BlockSpec((_NC, _R, _H), lambda i: (0, i, 0)),
        ],
        out_specs=[
            pl.BlockSpec((_R, _H), lambda i: (i, 0)),
            pl.BlockSpec((_R, _H), lambda i: (i, 0)),
            pl.BlockSpec((_R, 1), lambda i: (i, 0)),
        ],
        out_shape=[
            jax.ShapeDtypeStruct((_N, _H), jnp.float32),
            jax.ShapeDtypeStruct((_N, _H), jnp.float32),
            jax.ShapeDtypeStruct((_N, 1), jnp.float32),
        ],
    )(x, W_enc, Wp, deg2)


def _update(S, h, h0, dinv, w_ext, beta_arr, Wp):
    def body(s_ref, h_ref, h0_ref, dinv_ref, wext_ref, beta_ref, wp_ref,
             hn_ref, g_ref):
        sb = s_ref[...]
        hb = h_ref[...]
        dinv = dinv_ref[...]
        agg = dinv * (sb[0] + sb[1])
        out = agg - hb * wext_ref[...] + beta_ref[0, 0] * h0_ref[...]
        hn = hb + _STEP * jnp.maximum(out, 0.0)
        hn_ref[...] = hn
        g_ref[...] = dinv * jnp.dot(hn, wp_ref[...],
                                    preferred_element_type=jnp.float32)

    return pl.pallas_call(
        body,
        grid=(_NB,),
        in_specs=[
            pl.BlockSpec((_NC, _R, _H), lambda i: (0, i, 0)),
            pl.BlockSpec((_R, _H), lambda i: (i, 0)),
            pl.BlockSpec((_R, _H), lambda i: (i, 0)),
            pl.BlockSpec((_R, 1), lambda i: (i, 0)),
            pl.BlockSpec((1, _H), lambda i: (0, 0)),
            pl.BlockSpec((1, 1), lambda i: (0, 0)),
            pl.BlockSpec((_H, _H), lambda i: (0, 0)),
        ],
        out_specs=[
            pl.BlockSpec((_R, _H), lambda i: (i, 0)),
            pl.BlockSpec((_R, _H), lambda i: (i, 0)),
        ],
        out_shape=[
            jax.ShapeDtypeStruct((_N, _H), jnp.float32),
            jax.ShapeDtypeStruct((_N, _H), jnp.float32),
        ],
    )(S, h, h0, dinv, w_ext, beta_arr, Wp)


def _final(S, h, h0, dinv, w_ext, beta_arr, W_dec):
    def body(s_ref, h_ref, h0_ref, dinv_ref, wext_ref, beta_ref, wd_ref,
             y_ref):
        sb = s_ref[...]
        hb = h_ref[...]
        agg = dinv_ref[...] * (sb[0] + sb[1])
        out = agg - hb * wext_ref[...] + beta_ref[0, 0] * h0_ref[...]
        hn = hb + _STEP * jnp.maximum(out, 0.0)
        y_ref[...] = jnp.dot(hn, wd_ref[...], preferred_element_type=jnp.float32)

    return pl.pallas_call(
        body,
        grid=(_NB,),
        in_specs=[
            pl.BlockSpec((_NC, _R, _H), lambda i: (0, i, 0)),
            pl.BlockSpec((_R, _H), lambda i: (i, 0)),
            pl.BlockSpec((_R, _H), lambda i: (i, 0)),
            pl.BlockSpec((_R, 1), lambda i: (i, 0)),
            pl.BlockSpec((1, _H), lambda i: (0, 0)),
            pl.BlockSpec((1, 1), lambda i: (0, 0)),
            pl.BlockSpec((_H, _C), lambda i: (0, 0)),
        ],
        out_specs=pl.BlockSpec((_R, _C), lambda i: (i, 0)),
        out_shape=jax.ShapeDtypeStruct((_N, _C), jnp.float32),
    )(S, h, h0, dinv, w_ext, beta_arr, W_dec)


def kernel(x, edge_index, W_enc, W_pair, w_ext, beta, W_dec):
    row = edge_index[0]
    col = edge_index[1]
    beta_arr = jnp.asarray(beta, jnp.float32).reshape(1, 1)

    Wp = _wp_compute(W_pair, W_pair[:, :_H].T)
    deg2 = _sc_degree(col)
    h, g, dinv = _encode(x, W_enc, Wp, deg2)
    h0 = h
    for _ in range(_L - 1):
        S = _sc_gather_scatter(g, row, col)
        h, g = _update(S, h, h0, dinv, w_ext, beta_arr, Wp)
    S = _sc_gather_scatter(g, row, col)
    return _final(S, h, h0, dinv, w_ext, beta_arr, W_dec)


# speedup vs baseline: 10.1258x; 1.3093x over previous
import functools

import jax
import jax.numpy as jnp
from jax import lax
from jax.experimental import pallas as pl
from jax.experimental.pallas import tpu as pltpu
from jax.experimental.pallas import tpu_sc as plsc

_N = 10000
_E = 320000
_D = 128
_H = 128
_C = 40
_L = 4
_STEP = 0.1

_NC = 2
_NS = 16
_NW = _NC * _NS
_EPW = _E // _NW
_CH = 40
_NCHUNK = _EPW // _CH
_RC = 624
_ZC = 104
_TAIL = _N - _NS * _RC

_R = 1000
_NB = _N // _R


def _sc_degree(col):
    mesh = plsc.VectorSubcoreMesh(core_axis_name="c", subcore_axis_name="s")

    @functools.partial(
        pl.kernel,
        out_type=jax.ShapeDtypeStruct((_NC, _N, _H), jnp.float32),
        mesh=mesh,
        scratch_types=[
            pltpu.VMEM((_CH,), jnp.int32),
            pltpu.VMEM((_CH, _H), jnp.float32),
            pltpu.VMEM((_ZC, _H), jnp.float32),
            pltpu.VMEM_SHARED((_N, _H), jnp.float32),
        ],
    )
    def k(col_hbm, out_hbm, cidx, ones_v, zbuf, shared):
        c = lax.axis_index("c")
        s = lax.axis_index("s")
        wid = c * _NS + s

        @pl.loop(0, _CH)
        def _(i):
            @pl.loop(0, _H, step=16)
            def _(j):
                ones_v[i, pl.ds(j, 16)] = jnp.full((16,), 1.0, jnp.float32)

        @pl.loop(0, _ZC)
        def _(i):
            @pl.loop(0, _H, step=16)
            def _(j):
                zbuf[i, pl.ds(j, 16)] = jnp.zeros((16,), jnp.float32)

        @pl.loop(0, _RC // _ZC)
        def _(t):
            pltpu.sync_copy(zbuf, shared.at[pl.ds(s * _RC + t * _ZC, _ZC)])

        @pl.when(s == _NS - 1)
        def _():
            pltpu.sync_copy(zbuf.at[pl.ds(0, _TAIL)],
                            shared.at[pl.ds(_NS * _RC, _TAIL)])

        plsc.subcore_barrier()

        @pl.loop(0, _NCHUNK)
        def _(i):
            pltpu.sync_copy(col_hbm.at[pl.ds(wid * _EPW + i * _CH, _CH)], cidx)
            pltpu.sync_copy(ones_v, shared.at[cidx], add=True)

        plsc.subcore_barrier()

        @pl.loop(0, _RC // _ZC)
        def _(t):
            pltpu.sync_copy(
                shared.at[pl.ds(s * _RC + t * _ZC, _ZC)],
                out_hbm.at[c, pl.ds(s * _RC + t * _ZC, _ZC)],
            )

        @pl.when(s == _NS - 1)
        def _():
            pltpu.sync_copy(shared.at[pl.ds(_NS * _RC, _TAIL)],
                            out_hbm.at[c, pl.ds(_NS * _RC, _TAIL)])

    return k(col)


_NBUF = 5


def _sc_gather_scatter(g, eidx):
    mesh = plsc.VectorSubcoreMesh(core_axis_name="c", subcore_axis_name="s")

    @functools.partial(
        pl.kernel,
        out_type=jax.ShapeDtypeStruct((_NC, _N, _H), jnp.float32),
        mesh=mesh,
        scratch_types=(
            [pltpu.VMEM((2, _CH), jnp.int32) for _ in range(_NBUF)]
            + [pltpu.VMEM((_CH, _H), jnp.float32) for _ in range(_NBUF)]
            + [pltpu.VMEM((_ZC, _H), jnp.float32)]
            + [pltpu.VMEM_SHARED((_N, _H), jnp.float32)]
            + [pltpu.SemaphoreType.DMA for _ in range(_NBUF)]
        ),
    )
    def k(g_hbm, eidx_hbm, out_hbm, *refs):
        idxb = refs[:_NBUF]
        rows = refs[_NBUF:2 * _NBUF]
        zbuf = refs[2 * _NBUF]
        shared = refs[2 * _NBUF + 1]
        sems = refs[2 * _NBUF + 2:]

        c = lax.axis_index("c")
        s = lax.axis_index("s")
        wid = c * _NS + s
        cbase = wid * _NCHUNK

        for b in range(_NBUF):
            pltpu.sync_copy(eidx_hbm.at[cbase + b], idxb[b])
            pltpu.make_async_copy(g_hbm.at[idxb[b].at[0]], rows[b],
                                  sems[b]).start()

        @pl.loop(0, _ZC)
        def _(i):
            @pl.loop(0, _H, step=16)
            def _(j):
                zbuf[i, pl.ds(j, 16)] = jnp.zeros((16,), jnp.float32)

        @pl.loop(0, _RC // _ZC)
        def _(t):
            pltpu.sync_copy(zbuf, shared.at[pl.ds(s * _RC + t * _ZC, _ZC)])

        @pl.when(s == _NS - 1)
        def _():
            pltpu.sync_copy(zbuf.at[pl.ds(0, _TAIL)],
                            shared.at[pl.ds(_NS * _RC, _TAIL)])

        plsc.subcore_barrier()

        @pl.loop(0, _NCHUNK // _NBUF - 1)
        def _(t):
            for b in range(_NBUF):
                pltpu.make_async_copy(g_hbm.at[idxb[b].at[0]], rows[b],
                                      sems[b]).wait()
                pltpu.sync_copy(rows[b], shared.at[idxb[b].at[1]], add=True)
                nxt = cbase + (t + 1) * _NBUF + b
                pltpu.sync_copy(eidx_hbm.at[nxt], idxb[b])
                pltpu.make_async_copy(g_hbm.at[idxb[b].at[0]], rows[b],
                                      sems[b]).start()

        for b in range(_NBUF):
            pltpu.make_async_copy(g_hbm.at[idxb[b].at[0]], rows[b],
                                  sems[b]).wait()
            pltpu.sync_copy(rows[b], shared.at[idxb[b].at[1]], add=True)

        plsc.subcore_barrier()

        @pl.loop(0, _RC // _ZC)
        def _(t):
            pltpu.sync_copy(
                shared.at[pl.ds(s * _RC + t * _ZC, _ZC)],
                out_hbm.at[c, pl.ds(s * _RC + t * _ZC, _ZC)],
            )

        @pl.when(s == _NS - 1)
        def _():
            pltpu.sync_copy(shared.at[pl.ds(_NS * _RC, _TAIL)],
                            out_hbm.at[c, pl.ds(_NS * _RC, _TAIL)])

    return k(g, eidx)


def _wp_compute(W_pair, W_pair_T):
    def body(wp_ref, wpt_ref, out_ref):
        w = wp_ref[...]
        a_t = wpt_ref[...]
        a = w[:, :_H]
        q = w[:, _H:_H + 1]
        r = w[:, _H + 1:_H + 2]
        rows = lax.broadcasted_iota(jnp.int32, (_H, _H), 0)
        cols = lax.broadcasted_iota(jnp.int32, (_H, _H), 1)
        upper = jnp.where(cols > rows, a, 0.0)
        lower = jnp.where(rows > cols, a_t, 0.0)
        w0 = upper + lower
        sumabs = jnp.sum(jnp.abs(w0), axis=1, keepdims=True)
        diagv = q * sumabs + r
        out_ref[...] = w0 + jnp.where(rows == cols, diagv, 0.0)

    return pl.pallas_call(
        body,
        out_shape=jax.ShapeDtypeStruct((_H, _H), jnp.float32),
    )(W_pair, W_pair_T)


def _encode(x, W_enc, Wp, deg2):
    def body(x_ref, we_ref, wp_ref, deg_ref, h_ref, g_ref, dinv_ref):
        db = deg_ref[...]
        d = db[0, :, 0:1] + db[1, :, 0:1]
        dinv = jnp.where(d > 0, lax.rsqrt(d), 0.0)
        h = jnp.dot(x_ref[...], we_ref[...], preferred_element_type=jnp.float32)
        g = dinv * jnp.dot(h, wp_ref[...], preferred_element_type=jnp.float32)
        h_ref[...] = h
        g_ref[...] = g
        dinv_ref[...] = dinv

    return pl.pallas_call(
        body,
        grid=(_NB,),
        in_specs=[
            pl.BlockSpec((_R, _D), lambda i: (i, 0)),
            pl.BlockSpec((_D, _H), lambda i: (0, 0)),
            pl.BlockSpec((_H, _H), lambda i: (0, 0)),
            pl.BlockSpec((_NC, _R, _H), lambda i: (0, i, 0)),
        ],
        out_specs=[
            pl.BlockSpec((_R, _H), lambda i: (i, 0)),
            pl.BlockSpec((_R, _H), lambda i: (i, 0)),
            pl.BlockSpec((_R, 1), lambda i: (i, 0)),
        ],
        out_shape=[
            jax.ShapeDtypeStruct((_N, _H), jnp.float32),
            jax.ShapeDtypeStruct((_N, _H), jnp.float32),
            jax.ShapeDtypeStruct((_N, 1), jnp.float32),
        ],
    )(x, W_enc, Wp, deg2)


def _update(S, h, h0, dinv, w_ext, beta_arr, Wp):
    def body(s_ref, h_ref, h0_ref, dinv_ref, wext_ref, beta_ref, wp_ref,
             hn_ref, g_ref):
        sb = s_ref[...]
        hb = h_ref[...]
        dinv = dinv_ref[...]
        agg = dinv * (sb[0] + sb[1])
        out = agg - hb * wext_ref[...] + beta_ref[0, 0] * h0_ref[...]
        hn = hb + _STEP * jnp.maximum(out, 0.0)
        hn_ref[...] = hn
        g_ref[...] = dinv * jnp.dot(hn, wp_ref[...],
                                    preferred_element_type=jnp.float32)

    return pl.pallas_call(
        body,
        grid=(_NB,),
        in_specs=[
            pl.BlockSpec((_NC, _R, _H), lambda i: (0, i, 0)),
            pl.BlockSpec((_R, _H), lambda i: (i, 0)),
            pl.BlockSpec((_R, _H), lambda i: (i, 0)),
            pl.BlockSpec((_R, 1), lambda i: (i, 0)),
            pl.BlockSpec((1, _H), lambda i: (0, 0)),
            pl.BlockSpec((1, 1), lambda i: (0, 0)),
            pl.BlockSpec((_H, _H), lambda i: (0, 0)),
        ],
        out_specs=[
            pl.BlockSpec((_R, _H), lambda i: (i, 0)),
            pl.BlockSpec((_R, _H), lambda i: (i, 0)),
        ],
        out_shape=[
            jax.ShapeDtypeStruct((_N, _H), jnp.float32),
            jax.ShapeDtypeStruct((_N, _H), jnp.float32),
        ],
    )(S, h, h0, dinv, w_ext, beta_arr, Wp)


def _final(S, h, h0, dinv, w_ext, beta_arr, W_dec):
    def body(s_ref, h_ref, h0_ref, dinv_ref, wext_ref, beta_ref, wd_ref,
             y_ref):
        sb = s_ref[...]
        hb = h_ref[...]
        agg = dinv_ref[...] * (sb[0] + sb[1])
        out = agg - hb * wext_ref[...] + beta_ref[0, 0] * h0_ref[...]
        hn = hb + _STEP * jnp.maximum(out, 0.0)
        y_ref[...] = jnp.dot(hn, wd_ref[...], preferred_element_type=jnp.float32)

    return pl.pallas_call(
        body,
        grid=(_NB,),
        in_specs=[
            pl.BlockSpec((_NC, _R, _H), lambda i: (0, i, 0)),
            pl.BlockSpec((_R, _H), lambda i: (i, 0)),
            pl.BlockSpec((_R, _H), lambda i: (i, 0)),
            pl.BlockSpec((_R, 1), lambda i: (i, 0)),
            pl.BlockSpec((1, _H), lambda i: (0, 0)),
            pl.BlockSpec((1, 1), lambda i: (0, 0)),
            pl.BlockSpec((_H, _C), lambda i: (0, 0)),
        ],
        out_specs=pl.BlockSpec((_R, _C), lambda i: (i, 0)),
        out_shape=jax.ShapeDtypeStruct((_N, _C), jnp.float32),
    )(S, h, h0, dinv, w_ext, beta_arr, W_dec)


def kernel(x, edge_index, W_enc, W_pair, w_ext, beta, W_dec):
    col = edge_index[1]
    eidx = (edge_index.reshape(2, _NW, _NCHUNK, _CH)
            .transpose(1, 2, 0, 3)
            .reshape(_NW * _NCHUNK, 2, _CH))
    beta_arr = jnp.asarray(beta, jnp.float32).reshape(1, 1)

    Wp = _wp_compute(W_pair, W_pair[:, :_H].T)
    deg2 = _sc_degree(col)
    h, g, dinv = _encode(x, W_enc, Wp, deg2)
    h0 = h
    for _ in range(_L - 1):
        S = _sc_gather_scatter(g, eidx)
        h, g = _update(S, h, h0, dinv, w_ext, beta_arr, Wp)
    S = _sc_gather_scatter(g, eidx)
    return _final(S, h, h0, dinv, w_ext, beta_arr, W_dec)


# speedup vs baseline: 13.3251x; 1.3159x over previous
import functools

import jax
import jax.numpy as jnp
from jax import lax
from jax.experimental import pallas as pl
from jax.experimental.pallas import tpu as pltpu
from jax.experimental.pallas import tpu_sc as plsc

_N = 10000
_E = 320000
_D = 128
_H = 128
_C = 40
_L = 4
_STEP = 0.1

_NC = 2
_NS = 16
_NW = _NC * _NS
_EPW = _E // _NW
_CH = 40
_NCHUNK = _EPW // _CH
_RC = 624
_ZC = 104
_TAIL = _N - _NS * _RC

_R = 1000
_NB = _N // _R


def _sc_degree(col):
    mesh = plsc.VectorSubcoreMesh(core_axis_name="c", subcore_axis_name="s")

    @functools.partial(
        pl.kernel,
        out_type=jax.ShapeDtypeStruct((_NC, _N, _H), jnp.float32),
        mesh=mesh,
        scratch_types=[
            pltpu.VMEM((_CH,), jnp.int32),
            pltpu.VMEM((_CH, _H), jnp.float32),
            pltpu.VMEM((_ZC, _H), jnp.float32),
            pltpu.VMEM_SHARED((_N, _H), jnp.float32),
        ],
    )
    def k(col_hbm, out_hbm, cidx, ones_v, zbuf, shared):
        c = lax.axis_index("c")
        s = lax.axis_index("s")
        wid = c * _NS + s

        @pl.loop(0, _CH)
        def _(i):
            @pl.loop(0, _H, step=16)
            def _(j):
                ones_v[i, pl.ds(j, 16)] = jnp.full((16,), 1.0, jnp.float32)

        @pl.loop(0, _ZC)
        def _(i):
            @pl.loop(0, _H, step=16)
            def _(j):
                zbuf[i, pl.ds(j, 16)] = jnp.zeros((16,), jnp.float32)

        @pl.loop(0, _RC // _ZC)
        def _(t):
            pltpu.sync_copy(zbuf, shared.at[pl.ds(s * _RC + t * _ZC, _ZC)])

        @pl.when(s == _NS - 1)
        def _():
            pltpu.sync_copy(zbuf.at[pl.ds(0, _TAIL)],
                            shared.at[pl.ds(_NS * _RC, _TAIL)])

        plsc.subcore_barrier()

        @pl.loop(0, _NCHUNK)
        def _(i):
            pltpu.sync_copy(col_hbm.at[pl.ds(wid * _EPW + i * _CH, _CH)], cidx)
            pltpu.sync_copy(ones_v, shared.at[cidx], add=True)

        plsc.subcore_barrier()

        @pl.loop(0, _RC // _ZC)
        def _(t):
            pltpu.sync_copy(
                shared.at[pl.ds(s * _RC + t * _ZC, _ZC)],
                out_hbm.at[c, pl.ds(s * _RC + t * _ZC, _ZC)],
            )

        @pl.when(s == _NS - 1)
        def _():
            pltpu.sync_copy(shared.at[pl.ds(_NS * _RC, _TAIL)],
                            out_hbm.at[c, pl.ds(_NS * _RC, _TAIL)])

    return k(col)


_NBUF = 5


def _sc_gather_scatter(g, eidx):
    mesh = plsc.VectorSubcoreMesh(core_axis_name="c", subcore_axis_name="s")

    @functools.partial(
        pl.kernel,
        out_type=jax.ShapeDtypeStruct((_NC, _N, _H), jnp.float32),
        mesh=mesh,
        scratch_types=(
            [pltpu.VMEM((2, _CH), jnp.int32) for _ in range(_NBUF)]
            + [pltpu.VMEM((_CH, _H), jnp.float32) for _ in range(_NBUF)]
            + [pltpu.VMEM((_ZC, _H), jnp.float32)]
            + [pltpu.VMEM_SHARED((_N, _H), jnp.float32)]
            + [pltpu.SemaphoreType.DMA for _ in range(3 * _NBUF)]
        ),
    )
    def k(g_hbm, eidx_hbm, out_hbm, *refs):
        idxb = refs[:_NBUF]
        rows = refs[_NBUF:2 * _NBUF]
        zbuf = refs[2 * _NBUF]
        shared = refs[2 * _NBUF + 1]
        sems = refs[2 * _NBUF + 2:2 * _NBUF + 2 + _NBUF]
        ssems = refs[2 * _NBUF + 2 + _NBUF:2 * _NBUF + 2 + 2 * _NBUF]
        isems = refs[2 * _NBUF + 2 + 2 * _NBUF:]

        c = lax.axis_index("c")
        s = lax.axis_index("s")
        wid = c * _NS + s
        cbase = wid * _NCHUNK

        for b in range(_NBUF):
            pltpu.sync_copy(eidx_hbm.at[cbase + b], idxb[b])
            pltpu.make_async_copy(g_hbm.at[idxb[b].at[0]], rows[b],
                                  sems[b]).start()

        @pl.loop(0, _ZC)
        def _(i):
            @pl.loop(0, _H, step=16)
            def _(j):
                zbuf[i, pl.ds(j, 16)] = jnp.zeros((16,), jnp.float32)

        @pl.loop(0, _RC // _ZC)
        def _(t):
            pltpu.sync_copy(zbuf, shared.at[pl.ds(s * _RC + t * _ZC, _ZC)])

        @pl.when(s == _NS - 1)
        def _():
            pltpu.sync_copy(zbuf.at[pl.ds(0, _TAIL)],
                            shared.at[pl.ds(_NS * _RC, _TAIL)])

        plsc.subcore_barrier()

        @pl.loop(0, _NCHUNK // _NBUF - 1)
        def _(t):
            for b in range(_NBUF):
                pltpu.make_async_copy(g_hbm.at[idxb[b].at[0]], rows[b],
                                      sems[b]).wait()
                pltpu.make_async_copy(rows[b], shared.at[idxb[b].at[1]],
                                      ssems[b]).start(add=True)
            for b in range(_NBUF):
                pltpu.make_async_copy(rows[b], shared.at[idxb[b].at[1]],
                                      ssems[b]).wait()
                nxt = cbase + (t + 1) * _NBUF + b
                pltpu.make_async_copy(eidx_hbm.at[nxt], idxb[b],
                                      isems[b]).start()
            for b in range(_NBUF):
                pltpu.make_async_copy(eidx_hbm.at[cbase + b], idxb[b],
                                      isems[b]).wait()
                pltpu.make_async_copy(g_hbm.at[idxb[b].at[0]], rows[b],
                                      sems[b]).start()

        for b in range(_NBUF):
            pltpu.make_async_copy(g_hbm.at[idxb[b].at[0]], rows[b],
                                  sems[b]).wait()
            pltpu.sync_copy(rows[b], shared.at[idxb[b].at[1]], add=True)

        plsc.subcore_barrier()

        @pl.loop(0, _RC // _ZC)
        def _(t):
            pltpu.sync_copy(
                shared.at[pl.ds(s * _RC + t * _ZC, _ZC)],
                out_hbm.at[c, pl.ds(s * _RC + t * _ZC, _ZC)],
            )

        @pl.when(s == _NS - 1)
        def _():
            pltpu.sync_copy(shared.at[pl.ds(_NS * _RC, _TAIL)],
                            out_hbm.at[c, pl.ds(_NS * _RC, _TAIL)])

    return k(g, eidx)


def _wp_compute(W_pair, W_pair_T):
    def body(wp_ref, wpt_ref, out_ref):
        w = wp_ref[...]
        a_t = wpt_ref[...]
        a = w[:, :_H]
        q = w[:, _H:_H + 1]
        r = w[:, _H + 1:_H + 2]
        rows = lax.broadcasted_iota(jnp.int32, (_H, _H), 0)
        cols = lax.broadcasted_iota(jnp.int32, (_H, _H), 1)
        upper = jnp.where(cols > rows, a, 0.0)
        lower = jnp.where(rows > cols, a_t, 0.0)
        w0 = upper + lower
        sumabs = jnp.sum(jnp.abs(w0), axis=1, keepdims=True)
        diagv = q * sumabs + r
        out_ref[...] = w0 + jnp.where(rows == cols, diagv, 0.0)

    return pl.pallas_call(
        body,
        out_shape=jax.ShapeDtypeStruct((_H, _H), jnp.float32),
    )(W_pair, W_pair_T)


def _encode(x, W_enc, Wp, deg2):
    def body(x_ref, we_ref, wp_ref, deg_ref, h_ref, g_ref, dinv_ref):
        db = deg_ref[...]
        d = db[0, :, 0:1] + db[1, :, 0:1]
        dinv = jnp.where(d > 0, lax.rsqrt(d), 0.0)
        h = jnp.dot(x_ref[...], we_ref[...], preferred_element_type=jnp.float32)
        g = dinv * jnp.dot(h, wp_ref[...], preferred_element_type=jnp.float32)
        h_ref[...] = h
        g_ref[...] = g
        dinv_ref[...] = dinv

    return pl.pallas_call(
        body,
        grid=(_NB,),
        in_specs=[
            pl.BlockSpec((_R, _D), lambda i: (i, 0)),
            pl.BlockSpec((_D, _H), lambda i: (0, 0)),
            pl.BlockSpec((_H, _H), lambda i: (0, 0)),
            pl.BlockSpec((_NC, _R, _H), lambda i: (0, i, 0)),
        ],
        out_specs=[
            pl.BlockSpec((_R, _H), lambda i: (i, 0)),
            pl.BlockSpec((_R, _H), lambda i: (i, 0)),
            pl.BlockSpec((_R, 1), lambda i: (i, 0)),
        ],
        out_shape=[
            jax.ShapeDtypeStruct((_N, _H), jnp.float32),
            jax.ShapeDtypeStruct((_N, _H), jnp.float32),
            jax.ShapeDtypeStruct((_N, 1), jnp.float32),
        ],
    )(x, W_enc, Wp, deg2)


def _update(S, h, h0, dinv, w_ext, beta_arr, Wp):
    def body(s_ref, h_ref, h0_ref, dinv_ref, wext_ref, beta_ref, wp_ref,
             hn_ref, g_ref):
        sb = s_ref[...]
        hb = h_ref[...]
        dinv = dinv_ref[...]
        agg = dinv * (sb[0] + sb[1])
        out = agg - hb * wext_ref[...] + beta_ref[0, 0] * h0_ref[...]
        hn = hb + _STEP * jnp.maximum(out, 0.0)
        hn_ref[...] = hn
        g_ref[...] = dinv * jnp.dot(hn, wp_ref[...],
                                    preferred_element_type=jnp.float32)

    return pl.pallas_call(
        body,
        grid=(_NB,),
        in_specs=[
            pl.BlockSpec((_NC, _R, _H), lambda i: (0, i, 0)),
            pl.BlockSpec((_R, _H), lambda i: (i, 0)),
            pl.BlockSpec((_R, _H), lambda i: (i, 0)),
            pl.BlockSpec((_R, 1), lambda i: (i, 0)),
            pl.BlockSpec((1, _H), lambda i: (0, 0)),
            pl.BlockSpec((1, 1), lambda i: (0, 0)),
            pl.BlockSpec((_H, _H), lambda i: (0, 0)),
        ],
        out_specs=[
            pl.BlockSpec((_R, _H), lambda i: (i, 0)),
            pl.BlockSpec((_R, _H), lambda i: (i, 0)),
        ],
        out_shape=[
            jax.ShapeDtypeStruct((_N, _H), jnp.float32),
            jax.ShapeDtypeStruct((_N, _H), jnp.float32),
        ],
    )(S, h, h0, dinv, w_ext, beta_arr, Wp)


def _final(S, h, h0, dinv, w_ext, beta_arr, W_dec):
    def body(s_ref, h_ref, h0_ref, dinv_ref, wext_ref, beta_ref, wd_ref,
             y_ref):
        sb = s_ref[...]
        hb = h_ref[...]
        agg = dinv_ref[...] * (sb[0] + sb[1])
        out = agg - hb * wext_ref[...] + beta_ref[0, 0] * h0_ref[...]
        hn = hb + _STEP * jnp.maximum(out, 0.0)
        y_ref[...] = jnp.dot(hn, wd_ref[...], preferred_element_type=jnp.float32)

    return pl.pallas_call(
        body,
        grid=(_NB,),
        in_specs=[
            pl.BlockSpec((_NC, _R, _H), lambda i: (0, i, 0)),
            pl.BlockSpec((_R, _H), lambda i: (i, 0)),
            pl.BlockSpec((_R, _H), lambda i: (i, 0)),
            pl.BlockSpec((_R, 1), lambda i: (i, 0)),
            pl.BlockSpec((1, _H), lambda i: (0, 0)),
            pl.BlockSpec((1, 1), lambda i: (0, 0)),
            pl.BlockSpec((_H, _C), lambda i: (0, 0)),
        ],
        out_specs=pl.BlockSpec((_R, _C), lambda i: (i, 0)),
        out_shape=jax.ShapeDtypeStruct((_N, _C), jnp.float32),
    )(S, h, h0, dinv, w_ext, beta_arr, W_dec)


def kernel(x, edge_index, W_enc, W_pair, w_ext, beta, W_dec):
    col = edge_index[1]
    eidx = (edge_index.reshape(2, _NW, _NCHUNK, _CH)
            .transpose(1, 2, 0, 3)
            .reshape(_NW * _NCHUNK, 2, _CH))
    beta_arr = jnp.asarray(beta, jnp.float32).reshape(1, 1)

    Wp = _wp_compute(W_pair, W_pair[:, :_H].T)
    deg2 = _sc_degree(col)
    h, g, dinv = _encode(x, W_enc, Wp, deg2)
    h0 = h
    for _ in range(_L - 1):
        S = _sc_gather_scatter(g, eidx)
        h, g = _update(S, h, h0, dinv, w_ext, beta_arr, Wp)
    S = _sc_gather_scatter(g, eidx)
    return _final(S, h, h0, dinv, w_ext, beta_arr, W_dec)


# speedup vs baseline: 15.2430x; 1.1439x over previous
import functools

import jax
import jax.numpy as jnp
from jax import lax
from jax.experimental import pallas as pl
from jax.experimental.pallas import tpu as pltpu
from jax.experimental.pallas import tpu_sc as plsc

_N = 10000
_E = 320000
_D = 128
_H = 128
_C = 40
_L = 4
_STEP = 0.1

_NC = 2
_NS = 16
_NW = _NC * _NS
_EPW = _E // _NW

_CH = 50
_NCHUNK = _EPW // _CH
_NBUF = 5

_CHD = 80
_NCHD = _EPW // _CHD
_NBUFD = 5

_RC = 624
_ZC = 48
_TAIL = _N - _NS * _RC

_R = 1000
_NB = _N // _R


def _sc_degree(col):
    mesh = plsc.VectorSubcoreMesh(core_axis_name="c", subcore_axis_name="s")

    @functools.partial(
        pl.kernel,
        out_type=jax.ShapeDtypeStruct((_NC, _N, _H), jnp.float32),
        mesh=mesh,
        scratch_types=(
            [pltpu.VMEM((_CHD,), jnp.int32) for _ in range(_NBUFD)]
            + [pltpu.VMEM((_CHD, _H), jnp.float32)]
            + [pltpu.VMEM((_ZC, _H), jnp.float32)]
            + [pltpu.VMEM_SHARED((_N, _H), jnp.float32)]
            + [pltpu.SemaphoreType.DMA for _ in range(2 * _NBUFD)]
        ),
    )
    def k(col_hbm, out_hbm, *refs):
        cidx = refs[:_NBUFD]
        ones_v = refs[_NBUFD]
        zbuf = refs[_NBUFD + 1]
        shared = refs[_NBUFD + 2]
        ssems = refs[_NBUFD + 3:_NBUFD + 3 + _NBUFD]
        isems = refs[_NBUFD + 3 + _NBUFD:]

        c = lax.axis_index("c")
        s = lax.axis_index("s")
        wid = c * _NS + s
        ebase = wid * _EPW

        for b in range(_NBUFD):
            pltpu.sync_copy(col_hbm.at[pl.ds(ebase + b * _CHD, _CHD)], cidx[b])

        @pl.loop(0, _CHD)
        def _(i):
            @pl.loop(0, _H, step=16)
            def _(j):
                ones_v[i, pl.ds(j, 16)] = jnp.full((16,), 1.0, jnp.float32)

        @pl.loop(0, _ZC)
        def _(i):
            @pl.loop(0, _H, step=16)
            def _(j):
                zbuf[i, pl.ds(j, 16)] = jnp.zeros((16,), jnp.float32)

        @pl.loop(0, _RC // _ZC)
        def _(t):
            pltpu.sync_copy(zbuf, shared.at[pl.ds(s * _RC + t * _ZC, _ZC)])

        @pl.when(s == _NS - 1)
        def _():
            pltpu.sync_copy(zbuf.at[pl.ds(0, _TAIL)],
                            shared.at[pl.ds(_NS * _RC, _TAIL)])

        plsc.subcore_barrier()

        @pl.loop(0, _NCHD // _NBUFD - 1)
        def _(t):
            for b in range(_NBUFD):
                pltpu.make_async_copy(ones_v, shared.at[cidx[b]],
                                      ssems[b]).start(add=True)
            for b in range(_NBUFD):
                pltpu.make_async_copy(ones_v, shared.at[cidx[b]],
                                      ssems[b]).wait()
                nxt = ebase + ((t + 1) * _NBUFD + b) * _CHD
                pltpu.make_async_copy(col_hbm.at[pl.ds(nxt, _CHD)], cidx[b],
                                      isems[b]).start()
            for b in range(_NBUFD):
                pltpu.make_async_copy(col_hbm.at[pl.ds(ebase, _CHD)], cidx[b],
                                      isems[b]).wait()

        for b in range(_NBUFD):
            pltpu.sync_copy(ones_v, shared.at[cidx[b]], add=True)

        plsc.subcore_barrier()

        @pl.loop(0, _RC // _ZC)
        def _(t):
            pltpu.sync_copy(
                shared.at[pl.ds(s * _RC + t * _ZC, _ZC)],
                out_hbm.at[c, pl.ds(s * _RC + t * _ZC, _ZC)],
            )

        @pl.when(s == _NS - 1)
        def _():
            pltpu.sync_copy(shared.at[pl.ds(_NS * _RC, _TAIL)],
                            out_hbm.at[c, pl.ds(_NS * _RC, _TAIL)])

    return k(col)


def _sc_gather_scatter(g, eidx):
    mesh = plsc.VectorSubcoreMesh(core_axis_name="c", subcore_axis_name="s")

    @functools.partial(
        pl.kernel,
        out_type=jax.ShapeDtypeStruct((_NC, _N, _H), jnp.float32),
        mesh=mesh,
        scratch_types=(
            [pltpu.VMEM((2, _CH), jnp.int32) for _ in range(_NBUF)]
            + [pltpu.VMEM((_CH, _H), jnp.float32) for _ in range(_NBUF)]
            + [pltpu.VMEM((_ZC, _H), jnp.float32)]
            + [pltpu.VMEM_SHARED((_N, _H), jnp.float32)]
            + [pltpu.SemaphoreType.DMA for _ in range(3 * _NBUF)]
        ),
    )
    def k(g_hbm, eidx_hbm, out_hbm, *refs):
        idxb = refs[:_NBUF]
        rows = refs[_NBUF:2 * _NBUF]
        zbuf = refs[2 * _NBUF]
        shared = refs[2 * _NBUF + 1]
        sems = refs[2 * _NBUF + 2:2 * _NBUF + 2 + _NBUF]
        ssems = refs[2 * _NBUF + 2 + _NBUF:2 * _NBUF + 2 + 2 * _NBUF]
        isems = refs[2 * _NBUF + 2 + 2 * _NBUF:]

        c = lax.axis_index("c")
        s = lax.axis_index("s")
        wid = c * _NS + s
        cbase = wid * _NCHUNK

        for b in range(_NBUF):
            pltpu.sync_copy(eidx_hbm.at[cbase + b], idxb[b])
            pltpu.make_async_copy(g_hbm.at[idxb[b].at[0]], rows[b],
                                  sems[b]).start()

        @pl.loop(0, _ZC)
        def _(i):
            @pl.loop(0, _H, step=16)
            def _(j):
                zbuf[i, pl.ds(j, 16)] = jnp.zeros((16,), jnp.float32)

        @pl.loop(0, _RC // _ZC)
        def _(t):
            pltpu.sync_copy(zbuf, shared.at[pl.ds(s * _RC + t * _ZC, _ZC)])

        @pl.when(s == _NS - 1)
        def _():
            pltpu.sync_copy(zbuf.at[pl.ds(0, _TAIL)],
                            shared.at[pl.ds(_NS * _RC, _TAIL)])

        plsc.subcore_barrier()

        @pl.loop(0, _NCHUNK // _NBUF - 1)
        def _(t):
            for b in range(_NBUF):
                pltpu.make_async_copy(g_hbm.at[idxb[b].at[0]], rows[b],
                                      sems[b]).wait()
                pltpu.make_async_copy(rows[b], shared.at[idxb[b].at[1]],
                                      ssems[b]).start(add=True)
            for b in range(_NBUF):
                pltpu.make_async_copy(rows[b], shared.at[idxb[b].at[1]],
                                      ssems[b]).wait()
                nxt = cbase + (t + 1) * _NBUF + b
                pltpu.make_async_copy(eidx_hbm.at[nxt], idxb[b],
                                      isems[b]).start()
            for b in range(_NBUF):
                pltpu.make_async_copy(eidx_hbm.at[cbase + b], idxb[b],
                                      isems[b]).wait()
                pltpu.make_async_copy(g_hbm.at[idxb[b].at[0]], rows[b],
                                      sems[b]).start()

        for b in range(_NBUF):
            pltpu.make_async_copy(g_hbm.at[idxb[b].at[0]], rows[b],
                                  sems[b]).wait()
            pltpu.sync_copy(rows[b], shared.at[idxb[b].at[1]], add=True)

        plsc.subcore_barrier()

        @pl.loop(0, _RC // _ZC)
        def _(t):
            pltpu.sync_copy(
                shared.at[pl.ds(s * _RC + t * _ZC, _ZC)],
                out_hbm.at[c, pl.ds(s * _RC + t * _ZC, _ZC)],
            )

        @pl.when(s == _NS - 1)
        def _():
            pltpu.sync_copy(shared.at[pl.ds(_NS * _RC, _TAIL)],
                            out_hbm.at[c, pl.ds(_NS * _RC, _TAIL)])

    return k(g, eidx)


def _wp_compute(W_pair, W_pair_T):
    def body(wp_ref, wpt_ref, out_ref):
        w = wp_ref[...]
        a_t = wpt_ref[...]
        a = w[:, :_H]
        q = w[:, _H:_H + 1]
        r = w[:, _H + 1:_H + 2]
        rows = lax.broadcasted_iota(jnp.int32, (_H, _H), 0)
        cols = lax.broadcasted_iota(jnp.int32, (_H, _H), 1)
        upper = jnp.where(cols > rows, a, 0.0)
        lower = jnp.where(rows > cols, a_t, 0.0)
        w0 = upper + lower
        sumabs = jnp.sum(jnp.abs(w0), axis=1, keepdims=True)
        diagv = q * sumabs + r
        out_ref[...] = w0 + jnp.where(rows == cols, diagv, 0.0)

    return pl.pallas_call(
        body,
        out_shape=jax.ShapeDtypeStruct((_H, _H), jnp.float32),
    )(W_pair, W_pair_T)


def _encode_h(x, W_enc):
    def body(x_ref, we_ref, h_ref):
        h_ref[...] = jnp.dot(x_ref[...], we_ref[...],
                             preferred_element_type=jnp.float32)

    return pl.pallas_call(
        body,
        grid=(_NB,),
        in_specs=[
            pl.BlockSpec((_R, _D), lambda i: (i, 0)),
            pl.BlockSpec((_D, _H), lambda i: (0, 0)),
        ],
        out_specs=pl.BlockSpec((_R, _H), lambda i: (i, 0)),
        out_shape=jax.ShapeDtypeStruct((_N, _H), jnp.float32),
    )(x, W_enc)


def _encode_g(h, Wp, deg2):
    def body(h_ref, wp_ref, deg_ref, g_ref, dinv_ref):
        db = deg_ref[...]
        d = db[0, :, 0:1] + db[1, :, 0:1]
        dinv = jnp.where(d > 0, lax.rsqrt(d), 0.0)
        g = dinv * jnp.dot(h_ref[...], wp_ref[...],
                           preferred_element_type=jnp.float32)
        g_ref[...] = g
        dinv_ref[...] = dinv

    return pl.pallas_call(
        body,
        grid=(_NB,),
        in_specs=[
            pl.BlockSpec((_R, _H), lambda i: (i, 0)),
            pl.BlockSpec((_H, _H), lambda i: (0, 0)),
            pl.BlockSpec((_NC, _R, _H), lambda i: (0, i, 0)),
        ],
        out_specs=[
            pl.BlockSpec((_R, _H), lambda i: (i, 0)),
            pl.BlockSpec((_R, 1), lambda i: (i, 0)),
        ],
        out_shape=[
            jax.ShapeDtypeStruct((_N, _H), jnp.float32),
            jax.ShapeDtypeStruct((_N, 1), jnp.float32),
        ],
    )(h, Wp, deg2)


def _update(S, h, h0, dinv, w_ext, beta_arr, Wp):
    def body(s_ref, h_ref, h0_ref, dinv_ref, wext_ref, beta_ref, wp_ref,
             hn_ref, g_ref):
        sb = s_ref[...]
        hb = h_ref[...]
        dinv = dinv_ref[...]
        agg = dinv * (sb[0] + sb[1])
        out = agg - hb * wext_ref[...] + beta_ref[0, 0] * h0_ref[...]
        hn = hb + _STEP * jnp.maximum(out, 0.0)
        hn_ref[...] = hn
        g_ref[...] = dinv * jnp.dot(hn, wp_ref[...],
                                    preferred_element_type=jnp.float32)

    return pl.pallas_call(
        body,
        grid=(_NB,),
        in_specs=[
            pl.BlockSpec((_NC, _R, _H), lambda i: (0, i, 0)),
            pl.BlockSpec((_R, _H), lambda i: (i, 0)),
            pl.BlockSpec((_R, _H), lambda i: (i, 0)),
            pl.BlockSpec((_R, 1), lambda i: (i, 0)),
            pl.BlockSpec((1, _H), lambda i: (0, 0)),
            pl.BlockSpec((1, 1), lambda i: (0, 0)),
            pl.BlockSpec((_H, _H), lambda i: (0, 0)),
        ],
        out_specs=[
            pl.BlockSpec((_R, _H), lambda i: (i, 0)),
            pl.BlockSpec((_R, _H), lambda i: (i, 0)),
        ],
        out_shape=[
            jax.ShapeDtypeStruct((_N, _H), jnp.float32),
            jax.ShapeDtypeStruct((_N, _H), jnp.float32),
        ],
    )(S, h, h0, dinv, w_ext, beta_arr, Wp)


def _final(S, h, h0, dinv, w_ext, beta_arr, W_dec):
    def body(s_ref, h_ref, h0_ref, dinv_ref, wext_ref, beta_ref, wd_ref,
             y_ref):
        sb = s_ref[...]
        hb = h_ref[...]
        agg = dinv_ref[...] * (sb[0] + sb[1])
        out = agg - hb * wext_ref[...] + beta_ref[0, 0] * h0_ref[...]
        hn = hb + _STEP * jnp.maximum(out, 0.0)
        y_ref[...] = jnp.dot(hn, wd_ref[...], preferred_element_type=jnp.float32)

    return pl.pallas_call(
        body,
        grid=(_NB,),
        in_specs=[
            pl.BlockSpec((_NC, _R, _H), lambda i: (0, i, 0)),
            pl.BlockSpec((_R, _H), lambda i: (i, 0)),
            pl.BlockSpec((_R, _H), lambda i: (i, 0)),
            pl.BlockSpec((_R, 1), lambda i: (i, 0)),
            pl.BlockSpec((1, _H), lambda i: (0, 0)),
            pl.BlockSpec((1, 1), lambda i: (0, 0)),
            pl.BlockSpec((_H, _C), lambda i: (0, 0)),
        ],
        out_specs=pl.BlockSpec((_R, _C), lambda i: (i, 0)),
        out_shape=jax.ShapeDtypeStruct((_N, _C), jnp.float32),
    )(S, h, h0, dinv, w_ext, beta_arr, W_dec)


def kernel(x, edge_index, W_enc, W_pair, w_ext, beta, W_dec):
    col = edge_index[1]
    eidx = (edge_index.reshape(2, _NW, _NCHUNK, _CH)
            .transpose(1, 2, 0, 3)
            .reshape(_NW * _NCHUNK, 2, _CH))
    beta_arr = jnp.asarray(beta, jnp.float32).reshape(1, 1)

    Wp = _wp_compute(W_pair, W_pair[:, :_H].T)
    deg2 = _sc_degree(col)
    h = _encode_h(x, W_enc)
    g, dinv = _encode_g(h, Wp, deg2)
    h0 = h
    for _ in range(_L - 1):
        S = _sc_gather_scatter(g, eidx)
        h, g = _update(S, h, h0, dinv, w_ext, beta_arr, Wp)
    S = _sc_gather_scatter(g, eidx)
    return _final(S, h, h0, dinv, w_ext, beta_arr, W_dec)
